# baseline (speedup 1.0000x reference)
import functools

import jax
import jax.numpy as jnp
from jax import lax
from jax.experimental import pallas as pl
from jax.experimental.pallas import tpu as pltpu
from jax.experimental.pallas import tpu_sc as plsc

_QB = 400
_UNROLL = 50


def _block_argmax(sim):
    qb, b = sim.shape
    ns = qb // 8
    run_v = sim[0:8, :]
    run_r = jnp.zeros((8, b), jnp.float32)
    for r in range(1, ns):
        v = sim[r * 8:(r + 1) * 8, :]
        mask = v > run_v
        run_v = jnp.where(mask, v, run_v)
        run_r = jnp.where(mask, jnp.float32(r), run_r)
    srow = lax.broadcasted_iota(jnp.int32, (8, b), 0).astype(jnp.float32)
    grow = run_r * jnp.float32(8) + srow
    bm = jnp.max(run_v, axis=0)
    cand = jnp.where(run_v < bm[None, :], jnp.float32(3e38), grow)
    bi = jnp.min(cand, axis=0).astype(jnp.int32)
    return bm, bi


def _argmax_body(nq, qb, u, q_ref, xt_ref, idx_ref, rmax_ref, ridx_ref):
    i = pl.program_id(0)
    xt = xt_ref[...]
    for j in range(u):
        sim = lax.dot_general(
            q_ref[j * qb:(j + 1) * qb, :], xt,
            dimension_numbers=(((1,), (0,)), ((), ())),
            preferred_element_type=jnp.float32,
        )
        bm, bi = _block_argmax(sim)
        off = i * (u * qb) + j * qb

        if j == 0:
            @pl.when(i == 0)
            def _():
                rmax_ref[...] = bm
                ridx_ref[...] = bi

            @pl.when(i > 0)
            def _():
                prev_m = rmax_ref[...]
                prev_i = ridx_ref[...]
                better = bm > prev_m
                rmax_ref[...] = jnp.where(better, bm, prev_m)
                ridx_ref[...] = jnp.where(better, bi + off, prev_i)
        else:
            prev_m = rmax_ref[...]
            prev_i = ridx_ref[...]
            better = bm > prev_m
            rmax_ref[...] = jnp.where(better, bm, prev_m)
            ridx_ref[...] = jnp.where(better, bi + off, prev_i)

    @pl.when(i == nq - 1)
    def _():
        idx_ref[...] = ridx_ref[...]


def _nn_argmax(x, queue_x, qb=_QB, u=_UNROLL):
    b, d = x.shape
    n = queue_x.shape[0]
    nq = n // (qb * u)
    xt = x.T
    return pl.pallas_call(
        functools.partial(_argmax_body, nq, qb, u),
        grid=(nq,),
        in_specs=[
            pl.BlockSpec((qb * u, d), lambda i: (i, 0)),
            pl.BlockSpec((d, b), lambda i: (0, 0)),
        ],
        out_specs=pl.BlockSpec((b,), lambda i: (0,)),
        out_shape=jax.ShapeDtypeStruct((b,), jnp.int32),
        scratch_shapes=[
            pltpu.VMEM((b,), jnp.float32),
            pltpu.VMEM((b,), jnp.int32),
        ],
    )(queue_x, xt)


def _gather_rows(queue_x, idx):
    n, d = queue_x.shape
    b = idx.shape[0]
    info = plsc.get_sparse_core_info()
    nw = info.num_cores * info.num_subcores
    bpw = b // nw
    mesh = plsc.VectorSubcoreMesh(core_axis_name="c", subcore_axis_name="s")

    @functools.partial(
        pl.kernel,
        mesh=mesh,
        out_type=jax.ShapeDtypeStruct((b, d), jnp.float32),
        scratch_types=[
            pltpu.VMEM((bpw,), jnp.int32),
            pltpu.VMEM((bpw, d), jnp.float32),
            pltpu.SemaphoreType.DMA,
        ],
    )
    def gk(table_hbm, idx_hbm, out_hbm, idx_v, rows_v, sem):
        wid = lax.axis_index("s") * info.num_cores + lax.axis_index("c")
        base = wid * bpw
        pltpu.sync_copy(idx_hbm.at[pl.ds(base, bpw)], idx_v)
        pltpu.async_copy(table_hbm.at[idx_v], rows_v, sem).wait()
        pltpu.sync_copy(rows_v, out_hbm.at[pl.ds(base, bpw)])

    return gk(queue_x, idx)


def kernel(x, queue_x):
    idx = _nn_argmax(x, queue_x)
    return _gather_rows(queue_x, idx)


# speedup vs baseline: 1.2275x; 1.2275x over previous
import functools

import jax
import jax.numpy as jnp
from jax import lax
from jax.experimental import pallas as pl
from jax.experimental.pallas import tpu as pltpu
from jax.experimental.pallas import tpu_sc as plsc

_QB = 400
_UNROLL = 25


def _block_argmax(sim):
    qb, b = sim.shape
    ns = qb // 8
    run_v = sim[0:8, :]
    run_r = jnp.zeros((8, b), jnp.float32)
    for r in range(1, ns):
        v = sim[r * 8:(r + 1) * 8, :]
        mask = v > run_v
        run_v = jnp.where(mask, v, run_v)
        run_r = jnp.where(mask, jnp.float32(r), run_r)
    srow = lax.broadcasted_iota(jnp.int32, (8, b), 0).astype(jnp.float32)
    grow = run_r * jnp.float32(8) + srow
    bm = jnp.max(run_v, axis=0)
    cand = jnp.where(run_v < bm[None, :], jnp.float32(3e38), grow)
    bi = jnp.min(cand, axis=0).astype(jnp.int32)
    return bm, bi


def _argmax_body(nq, qb, u, q_ref, xt_ref, idx_ref, rmax_ref, ridx_ref):
    i = pl.program_id(0)
    xt = xt_ref[...]
    for j in range(u):
        sim = lax.dot_general(
            q_ref[j * qb:(j + 1) * qb, :], xt,
            dimension_numbers=(((1,), (0,)), ((), ())),
            preferred_element_type=jnp.float32,
        )
        bm, bi = _block_argmax(sim)
        off = i * (u * qb) + j * qb

        if j == 0:
            @pl.when(i == 0)
            def _():
                rmax_ref[...] = bm
                ridx_ref[...] = bi

            @pl.when(i > 0)
            def _():
                prev_m = rmax_ref[...]
                prev_i = ridx_ref[...]
                better = bm > prev_m
                rmax_ref[...] = jnp.where(better, bm, prev_m)
                ridx_ref[...] = jnp.where(better, bi + off, prev_i)
        else:
            prev_m = rmax_ref[...]
            prev_i = ridx_ref[...]
            better = bm > prev_m
            rmax_ref[...] = jnp.where(better, bm, prev_m)
            ridx_ref[...] = jnp.where(better, bi + off, prev_i)

    @pl.when(i == nq - 1)
    def _():
        idx_ref[...] = ridx_ref[...]


def _nn_argmax(x, queue_x, qb=_QB, u=_UNROLL):
    b, d = x.shape
    n = queue_x.shape[0]
    nq = n // (qb * u)
    xt = x.T
    return pl.pallas_call(
        functools.partial(_argmax_body, nq, qb, u),
        grid=(nq,),
        in_specs=[
            pl.BlockSpec((qb * u, d), lambda i: (i, 0)),
            pl.BlockSpec((d, b), lambda i: (0, 0)),
        ],
        out_specs=pl.BlockSpec((b,), lambda i: (0,)),
        out_shape=jax.ShapeDtypeStruct((b,), jnp.int32),
        scratch_shapes=[
            pltpu.VMEM((b,), jnp.float32),
            pltpu.VMEM((b,), jnp.int32),
        ],
    )(queue_x, xt)


def _gather_rows(queue_x, idx):
    n, d = queue_x.shape
    b = idx.shape[0]
    info = plsc.get_sparse_core_info()
    nw = info.num_cores * info.num_subcores
    bpw = b // nw
    mesh = plsc.VectorSubcoreMesh(core_axis_name="c", subcore_axis_name="s")

    @functools.partial(
        pl.kernel,
        mesh=mesh,
        out_type=jax.ShapeDtypeStruct((b, d), jnp.float32),
        scratch_types=[
            pltpu.VMEM((bpw,), jnp.int32),
            pltpu.VMEM((bpw, d), jnp.float32),
            pltpu.SemaphoreType.DMA,
        ],
    )
    def gk(table_hbm, idx_hbm, out_hbm, idx_v, rows_v, sem):
        wid = lax.axis_index("s") * info.num_cores + lax.axis_index("c")
        base = wid * bpw
        pltpu.sync_copy(idx_hbm.at[pl.ds(base, bpw)], idx_v)
        pltpu.async_copy(table_hbm.at[idx_v], rows_v, sem).wait()
        pltpu.sync_copy(rows_v, out_hbm.at[pl.ds(base, bpw)])

    return gk(queue_x, idx)


def kernel(x, queue_x):
    h = x.shape[0] // 2
    idx1 = _nn_argmax(x[:h], queue_x)
    out1 = _gather_rows(queue_x, idx1)
    idx2 = _nn_argmax(x[h:], queue_x)
    out2 = _gather_rows(queue_x, idx2)
    return jnp.concatenate([out1, out2], axis=0)


# speedup vs baseline: 1.2926x; 1.0530x over previous
import functools

import jax
import jax.numpy as jnp
from jax import lax
from jax.experimental import pallas as pl
from jax.experimental.pallas import tpu as pltpu
from jax.experimental.pallas import tpu_sc as plsc

_QB = 400
_UNROLL = 25


def _block_argmax(sim):
    qb, b = sim.shape
    ns = qb // 8
    run_v = sim[0:8, :]
    run_r = jnp.zeros((8, b), jnp.float32)
    for r in range(1, ns):
        v = sim[r * 8:(r + 1) * 8, :]
        mask = v > run_v
        run_v = jnp.where(mask, v, run_v)
        run_r = jnp.where(mask, jnp.float32(r), run_r)
    srow = lax.broadcasted_iota(jnp.int32, (8, b), 0).astype(jnp.float32)
    grow = run_r * jnp.float32(8) + srow
    bm = jnp.max(run_v, axis=0)
    cand = jnp.where(run_v < bm[None, :], jnp.float32(3e38), grow)
    bi = jnp.min(cand, axis=0).astype(jnp.int32)
    return bm, bi


def _argmax_body(nq, qb, u, q_ref, xt_ref, idx_ref, rmax_ref, ridx_ref):
    i = pl.program_id(0)
    xt = xt_ref[...]
    for j in range(u):
        sim = lax.dot_general(
            q_ref[j * qb:(j + 1) * qb, :], xt,
            dimension_numbers=(((1,), (0,)), ((), ())),
            preferred_element_type=jnp.float32,
        )
        bm, bi = _block_argmax(sim)
        off = i * (u * qb) + j * qb

        if j == 0:
            @pl.when(i == 0)
            def _():
                rmax_ref[...] = bm
                ridx_ref[...] = bi

            @pl.when(i > 0)
            def _():
                prev_m = rmax_ref[...]
                prev_i = ridx_ref[...]
                better = bm > prev_m
                rmax_ref[...] = jnp.where(better, bm, prev_m)
                ridx_ref[...] = jnp.where(better, bi + off, prev_i)
        else:
            prev_m = rmax_ref[...]
            prev_i = ridx_ref[...]
            better = bm > prev_m
            rmax_ref[...] = jnp.where(better, bm, prev_m)
            ridx_ref[...] = jnp.where(better, bi + off, prev_i)

    @pl.when(i == nq - 1)
    def _():
        idx_ref[...] = ridx_ref[...]


def _nn_argmax(x, queue_x, qb=_QB, u=_UNROLL):
    b, d = x.shape
    n = queue_x.shape[0]
    nq = n // (qb * u)
    xt = x.T
    return pl.pallas_call(
        functools.partial(_argmax_body, nq, qb, u),
        grid=(nq,),
        in_specs=[
            pl.BlockSpec((qb * u, d), lambda i: (i, 0)),
            pl.BlockSpec((d, b), lambda i: (0, 0)),
        ],
        out_specs=pl.BlockSpec((b,), lambda i: (0,)),
        out_shape=jax.ShapeDtypeStruct((b,), jnp.int32),
        scratch_shapes=[
            pltpu.VMEM((b,), jnp.float32),
            pltpu.VMEM((b,), jnp.int32),
        ],
    )(queue_x, xt)


def _gather_rows(queue_x, idx):
    n, d = queue_x.shape
    b = idx.shape[0]
    info = plsc.get_sparse_core_info()
    nw = info.num_cores * info.num_subcores
    bpw = b // nw
    mesh = plsc.VectorSubcoreMesh(core_axis_name="c", subcore_axis_name="s")

    @functools.partial(
        pl.kernel,
        mesh=mesh,
        out_type=jax.ShapeDtypeStruct((b, d), jnp.float32),
        scratch_types=[
            pltpu.VMEM((bpw,), jnp.int32),
            pltpu.VMEM((bpw, d), jnp.float32),
            pltpu.SemaphoreType.DMA,
        ],
    )
    def gk(table_hbm, idx_hbm, out_hbm, idx_v, rows_v, sem):
        wid = lax.axis_index("s") * info.num_cores + lax.axis_index("c")
        base = wid * bpw
        pltpu.sync_copy(idx_hbm.at[pl.ds(base, bpw)], idx_v)
        pltpu.async_copy(table_hbm.at[idx_v], rows_v, sem).wait()
        pltpu.sync_copy(rows_v, out_hbm.at[pl.ds(base, bpw)])

    return gk(queue_x, idx)


def kernel(x, queue_x):
    idx = _nn_argmax(x, queue_x)
    return _gather_rows(queue_x, idx)
